# baseline (speedup 1.0000x reference)
import functools

import jax
import jax.numpy as jnp
from jax import lax
from jax.experimental import pallas as pl
from jax.experimental.pallas import tpu as pltpu
from jax.experimental.pallas import tpu_sc as plsc

_INFO = plsc.get_sparse_core_info()
_NC = _INFO.num_cores
_NS = _INFO.num_subcores
_NW = _NC * _NS
_CH = 128


@functools.lru_cache(maxsize=None)
def _make_gather(vocab, dim, batch):
    b_per_w = batch // _NW
    nch = b_per_w // _CH
    mesh = plsc.VectorSubcoreMesh(core_axis_name="c", subcore_axis_name="s")

    @functools.partial(
        pl.kernel,
        mesh=mesh,
        compiler_params=pltpu.CompilerParams(use_tc_tiling_on_sc=False),
        out_type=jax.ShapeDtypeStruct((_NW, nch, _CH, dim), jnp.float32),
        scratch_types=[
            pltpu.VMEM((nch, _CH), jnp.int32),
            pltpu.VMEM((nch, _CH, dim), jnp.float32),
            pltpu.SemaphoreType.DMA,
        ],
    )
    def gather(table_hbm, idx_hbm, out_hbm, idx_v, rows_v, sem):
        wid = lax.axis_index("s") * _NC + lax.axis_index("c")
        pltpu.sync_copy(idx_hbm.at[wid], idx_v)
        copies = [
            pltpu.async_copy(table_hbm.at[idx_v.at[j]], rows_v.at[j], sem)
            for j in range(nch)
        ]
        for c in copies:
            c.wait()
        pltpu.sync_copy(rows_v, out_hbm.at[wid])

    return gather


def kernel(class_label, table):
    batch = class_label.shape[0]
    vocab, dim = table.shape
    idx = class_label.astype(jnp.int32).reshape(_NW, batch // (_NW * _CH), _CH)
    out = _make_gather(vocab, dim, batch)(table, idx)
    return out.reshape(batch, 1, dim)


# speedup vs baseline: 2.4707x; 2.4707x over previous
import functools

import jax
import jax.numpy as jnp
from jax import lax
from jax.experimental import pallas as pl
from jax.experimental.pallas import tpu as pltpu
from jax.experimental.pallas import tpu_sc as plsc

_INFO = plsc.get_sparse_core_info()
_NC = _INFO.num_cores
_NS = _INFO.num_subcores
_NW = _NC * _NS
_NBUF = 4
_LANES = 16


@functools.lru_cache(maxsize=None)
def _make_gather(vocab, dim, batch):
    b_per_w = batch // _NW
    blk = b_per_w * dim
    nquad = b_per_w // _NBUF
    mesh = plsc.VectorSubcoreMesh(core_axis_name="c", subcore_axis_name="s")

    @functools.partial(
        pl.kernel,
        mesh=mesh,
        compiler_params=pltpu.CompilerParams(needs_layout_passes=False),
        out_type=jax.ShapeDtypeStruct((batch * dim,), jnp.float32),
        scratch_types=[
            pltpu.VMEM((b_per_w + _LANES,), jnp.int32),
            pltpu.VMEM((_NBUF, dim, 128), jnp.float32),
            pltpu.VMEM((blk,), jnp.float32),
            [pltpu.SemaphoreType.DMA] * _NBUF,
        ],
    )
    def gather(table_t_hbm, idx_hbm, out_hbm, idx_v, blks_v, rows_v, sems):
        wid = lax.axis_index("s") * _NC + lax.axis_index("c")
        pltpu.sync_copy(idx_hbm.at[wid], idx_v.at[pl.ds(0, b_per_w)])

        def fire(j, col_scalar):
            base = pl.multiple_of(col_scalar & ~127, 128)
            pltpu.async_copy(
                table_t_hbm.at[:, pl.ds(base, 128)], blks_v.at[j], sems[j]
            )

        def extract(j, col_scalar, i_local):
            lane = jnp.broadcast_to(col_scalar & 127, (_LANES,))
            for u in range(dim // _LANES):
                rows = lax.iota(jnp.int32, _LANES) + (u * _LANES)
                vals = plsc.load_gather(blks_v.at[j], [rows, lane])
                rows_v[pl.ds(i_local * dim + u * _LANES, _LANES)] = vals

        vec0 = idx_v[pl.ds(0, _LANES)]
        for j in range(_NBUF):
            fire(j, vec0[j])

        def body(q, prev):
            cur = idx_v[pl.ds(q * _NBUF, _LANES)]
            for j in range(_NBUF):
                pltpu.make_async_copy(
                    table_t_hbm.at[:, pl.ds(0, 128)], blks_v.at[j], sems[j]
                ).wait()
                extract(j, prev[j], (q - 1) * _NBUF + j)
                fire(j, cur[j])
            return cur

        last = lax.fori_loop(1, nquad, body, vec0)

        for j in range(_NBUF):
            pltpu.make_async_copy(
                table_t_hbm.at[:, pl.ds(0, 128)], blks_v.at[j], sems[j]
            ).wait()
            extract(j, last[j], (nquad - 1) * _NBUF + j)

        pltpu.sync_copy(rows_v, out_hbm.at[pl.ds(wid * blk, blk)])

    return gather


def kernel(class_label, table):
    batch = class_label.shape[0]
    vocab, dim = table.shape
    idx = class_label.astype(jnp.int32).reshape(_NW, batch // _NW)
    out = _make_gather(vocab, dim, batch)(table.T, idx)
    return out.reshape(batch, 1, dim)


# speedup vs baseline: 2.8734x; 1.1630x over previous
import functools

import jax
import jax.numpy as jnp
from jax import lax
from jax.experimental import pallas as pl
from jax.experimental.pallas import tpu as pltpu
from jax.experimental.pallas import tpu_sc as plsc

_INFO = plsc.get_sparse_core_info()
_NC = _INFO.num_cores
_NS = _INFO.num_subcores
_NW = _NC * _NS
_NBUF = 8
_LANES = 16


@functools.lru_cache(maxsize=None)
def _make_gather(vocab, dim, batch):
    b_per_w = batch // _NW
    blk = b_per_w * dim
    nquad = b_per_w // _NBUF
    mesh = plsc.VectorSubcoreMesh(core_axis_name="c", subcore_axis_name="s")

    @functools.partial(
        pl.kernel,
        mesh=mesh,
        compiler_params=pltpu.CompilerParams(needs_layout_passes=False),
        out_type=jax.ShapeDtypeStruct((batch * dim,), jnp.float32),
        scratch_types=[
            pltpu.VMEM((b_per_w + _LANES,), jnp.int32),
            pltpu.VMEM((_NBUF, dim, 128), jnp.float32),
            pltpu.VMEM((blk,), jnp.float32),
            [pltpu.SemaphoreType.DMA] * _NBUF,
        ],
    )
    def gather(table_t_hbm, idx_hbm, out_hbm, idx_v, blks_v, rows_v, sems):
        wid = lax.axis_index("s") * _NC + lax.axis_index("c")
        pltpu.sync_copy(idx_hbm.at[wid], idx_v.at[pl.ds(0, b_per_w)])

        def fire(j, col_scalar):
            base = pl.multiple_of(col_scalar & ~127, 128)
            pltpu.async_copy(
                table_t_hbm.at[:, pl.ds(base, 128)], blks_v.at[j], sems[j]
            )

        def extract(j, col_scalar, i_local):
            lane = jnp.broadcast_to(col_scalar & 127, (_LANES,))
            for u in range(dim // _LANES):
                rows = lax.iota(jnp.int32, _LANES) + (u * _LANES)
                vals = plsc.load_gather(blks_v.at[j], [rows, lane])
                rows_v[pl.ds(i_local * dim + u * _LANES, _LANES)] = vals

        vec0 = idx_v[pl.ds(0, _LANES)]
        for j in range(_NBUF):
            fire(j, vec0[j])

        def body(q, prev):
            cur = idx_v[pl.ds(q * _NBUF, _LANES)]
            for j in range(_NBUF):
                pltpu.make_async_copy(
                    table_t_hbm.at[:, pl.ds(0, 128)], blks_v.at[j], sems[j]
                ).wait()
                extract(j, prev[j], (q - 1) * _NBUF + j)
                fire(j, cur[j])
            return cur

        last = lax.fori_loop(1, nquad, body, vec0)

        for j in range(_NBUF):
            pltpu.make_async_copy(
                table_t_hbm.at[:, pl.ds(0, 128)], blks_v.at[j], sems[j]
            ).wait()
            extract(j, last[j], (nquad - 1) * _NBUF + j)

        pltpu.sync_copy(rows_v, out_hbm.at[pl.ds(wid * blk, blk)])

    return gather


def kernel(class_label, table):
    batch = class_label.shape[0]
    vocab, dim = table.shape
    idx = class_label.astype(jnp.int32).reshape(_NW, batch // _NW)
    out = _make_gather(vocab, dim, batch)(table.T, idx)
    return out.reshape(batch, 1, dim)
